# baseline (speedup 1.0000x reference)
import functools

import jax
import jax.numpy as jnp
import numpy as np
from jax import lax
from jax.experimental import pallas as pl
from jax.experimental.pallas import tpu as pltpu
from jax.experimental.pallas import tpu_sc as plsc

_ASD = np.array([
    [0,1,1,0,0,0,0,0,0,0,0,0,0],
    [0,0,0,1,1,0,0,0,0,0,0,0,0],
    [0,0,0,0,1,1,0,0,0,0,0,0,0],
    [0,0,0,0,0,0,1,1,0,0,0,0,0],
    [0,0,0,0,0,0,0,1,1,0,0,0,0],
    [0,0,0,0,0,0,0,0,1,1,0,0,0],
    [0,0,0,0,0,0,0,0,0,0,1,0,0],
    [0,0,0,0,0,0,0,0,0,0,1,1,0],
    [0,0,0,0,0,0,0,0,0,0,0,1,1],
    [0,0,0,0,0,0,0,0,0,0,0,0,1],
    [0,0,0,0,0,0,0,0,0,0,1,0,0],
    [0,0,0,0,0,0,0,0,0,0,0,1,0],
    [0,0,0,0,0,0,0,0,0,0,0,0,1]], dtype=np.float32)

_NC = 2
_NS = 16
_NW = _NC * _NS


def _policy_body(sub_ref, dip_ref, idx_ref, dmask_ref, asd_ref,
                 pol_ref, sv_ref, fidx_ref):
    B = sub_ref.shape[0]
    l = sub_ref[...]
    row = lax.broadcasted_iota(jnp.int32, l.shape, 1)
    l = jnp.where(row == 0, 1.0, l)
    m = asd_ref[...] > 0.0
    e = jnp.where(m[None, :, :], jnp.exp(l), 0.0)
    pol_ref[...] = e / jnp.sum(e, axis=2, keepdims=True)

    d = dip_ref[...]
    de = jnp.where(dmask_ref[...] > 0.0, jnp.exp(d), 0.0)
    sv_ref[...] = de / jnp.sum(de, axis=1, keepdims=True)

    idx = idx_ref[...]
    b_global = pl.program_id(0) * B + lax.broadcasted_iota(jnp.int32, idx.shape, 0)
    fidx_ref[...] = idx + 13 * b_global


def _sc_gather_body(bpt, chb, S, table_hbm, fidx_hbm, r0_hbm, c0_hbm,
                    out_hbm, idx_v, rows_v, comp_v, r0_v, c0_v, sem):
    wid = lax.axis_index("s") * _NC + lax.axis_index("c")
    b00 = wid * bpt
    rows_chunk = chb * S
    pltpu.sync_copy(r0_hbm, r0_v)
    pltpu.sync_copy(c0_hbm, c0_v)

    def do_chunk(g, carry):
        b0 = b00 + g * chb
        base = b0 * S
        pltpu.sync_copy(fidx_hbm.at[pl.ds(base, rows_chunk)], idx_v)
        hs = []
        for j in range(rows_chunk // 128):
            hs.append(pltpu.async_copy(
                table_hbm.at[idx_v.at[pl.ds(j * 128, 128)]],
                rows_v.at[pl.ds(j * 128, 128)], sem))
        for h in hs:
            h.wait()

        def compact(q, c):
            q16 = q * 16
            for m in range(13):
                rvec = r0_v[m] + q16
                v = plsc.load_gather(rows_v, [rvec, c0_v[m]])
                comp_v[pl.ds((q * 13 + m) * 16, 16)] = v
            return c

        lax.fori_loop(0, rows_chunk // 16, compact, 0)
        pltpu.sync_copy(comp_v.at[pl.ds(0, rows_chunk * 13)],
                        out_hbm.at[pl.ds(base * 13, rows_chunk * 13)])
        return carry

    lax.fori_loop(0, bpt // chb, do_chunk, 0)


def kernel(sub_logit, dip_logit, sub_pos_samples, dip_mask):
    N, K, _ = sub_logit.shape
    S = sub_pos_samples.shape[1]
    B = 256
    asd = jnp.asarray(_ASD)
    dmaskf = dip_mask.astype(jnp.float32)
    pol, sv, fidx = pl.pallas_call(
        _policy_body,
        grid=(N // B,),
        in_specs=[
            pl.BlockSpec((B, K, K), lambda i: (i, 0, 0)),
            pl.BlockSpec((B, 3), lambda i: (i, 0)),
            pl.BlockSpec((B, S), lambda i: (i, 0)),
            pl.BlockSpec((B, 3), lambda i: (i, 0)),
            pl.BlockSpec((K, K), lambda i: (0, 0)),
        ],
        out_specs=[
            pl.BlockSpec((B, K, K), lambda i: (i, 0, 0)),
            pl.BlockSpec((B, 3), lambda i: (i, 0)),
            pl.BlockSpec((B, S), lambda i: (i, 0)),
        ],
        out_shape=[
            jax.ShapeDtypeStruct((N, K, K), jnp.float32),
            jax.ShapeDtypeStruct((N, 3), jnp.float32),
            jax.ShapeDtypeStruct((N, S), jnp.int32),
        ],
    )(sub_logit, dip_logit, sub_pos_samples, dmaskf, asd)

    table16 = jnp.pad(pol.reshape(N * K, K), ((0, 0), (0, 16 - K)))

    rows = N * S
    bpt = N // _NW
    chb = 16
    mesh = plsc.VectorSubcoreMesh(core_axis_name="c", subcore_axis_name="s",
                                  num_cores=_NC, num_subcores=_NS)
    o = np.arange(13 * 16, dtype=np.int32)
    r0 = jnp.asarray(((o // 13) * 1).reshape(13, 16))
    c0 = jnp.asarray((o % 13).reshape(13, 16))
    samp_flat = pl.kernel(
        functools.partial(_sc_gather_body, bpt, chb, S),
        out_type=jax.ShapeDtypeStruct((rows * K,), jnp.float32),
        mesh=mesh,
        scratch_types=[
            pltpu.VMEM((chb * S,), jnp.int32),
            pltpu.VMEM((chb * S, 16), jnp.float32),
            pltpu.VMEM((chb * S * K,), jnp.float32),
            pltpu.VMEM((13, 16), jnp.int32),
            pltpu.VMEM((13, 16), jnp.int32),
            pltpu.SemaphoreType.DMA,
        ],
        compiler_params=pltpu.CompilerParams(use_tc_tiling_on_sc=False,
                                             needs_layout_passes=False),
    )(table16, fidx.reshape(rows), r0, c0)
    return (pol, samp_flat.reshape(N, S, K), sv)


# speedup vs baseline: 1.1687x; 1.1687x over previous
import functools

import jax
import jax.numpy as jnp
import numpy as np
from jax import lax
from jax.experimental import pallas as pl
from jax.experimental.pallas import tpu as pltpu
from jax.experimental.pallas import tpu_sc as plsc

_ASD = np.array([
    [0,1,1,0,0,0,0,0,0,0,0,0,0],
    [0,0,0,1,1,0,0,0,0,0,0,0,0],
    [0,0,0,0,1,1,0,0,0,0,0,0,0],
    [0,0,0,0,0,0,1,1,0,0,0,0,0],
    [0,0,0,0,0,0,0,1,1,0,0,0,0],
    [0,0,0,0,0,0,0,0,1,1,0,0,0],
    [0,0,0,0,0,0,0,0,0,0,1,0,0],
    [0,0,0,0,0,0,0,0,0,0,1,1,0],
    [0,0,0,0,0,0,0,0,0,0,0,1,1],
    [0,0,0,0,0,0,0,0,0,0,0,0,1],
    [0,0,0,0,0,0,0,0,0,0,1,0,0],
    [0,0,0,0,0,0,0,0,0,0,0,1,0],
    [0,0,0,0,0,0,0,0,0,0,0,0,1]], dtype=np.float32)

_NC = 2
_NS = 16
_NW = _NC * _NS


def _policy_body(sub_ref, dip_ref, idx_ref, dmask_ref, asd_ref,
                 pol_ref, sv_ref, fidx_ref):
    B = sub_ref.shape[0]
    l = sub_ref[...]
    row = lax.broadcasted_iota(jnp.int32, l.shape, 1)
    l = jnp.where(row == 0, 1.0, l)
    m = asd_ref[...] > 0.0
    e = jnp.where(m[None, :, :], jnp.exp(l), 0.0)
    pol_ref[...] = e / jnp.sum(e, axis=2, keepdims=True)

    d = dip_ref[...]
    de = jnp.where(dmask_ref[...] > 0.0, jnp.exp(d), 0.0)
    sv_ref[...] = de / jnp.sum(de, axis=1, keepdims=True)

    idx = idx_ref[...]
    b_global = pl.program_id(0) * B + lax.broadcasted_iota(jnp.int32, idx.shape, 0)
    fidx_ref[...] = idx + 13 * b_global


def _sc_gather_body(bpt, chb, S, table_hbm, fidx_hbm, out_hbm,
                    idx_v, rows_v, sem, osem):
    wid = lax.axis_index("s") * _NC + lax.axis_index("c")
    b00 = wid * bpt
    rows_chunk = chb * S

    def do_chunk(g, carry):
        b0 = b00 + g * chb
        base = b0 * S
        pltpu.sync_copy(fidx_hbm.at[pl.ds(base, rows_chunk)], idx_v)
        hs = []
        for j in range(rows_chunk // 128):
            hs.append(pltpu.async_copy(
                table_hbm.at[idx_v.at[pl.ds(j * 128, 128)]],
                rows_v.at[pl.ds(j * 128, 128)], sem))
        for h in hs:
            h.wait()
        os_ = []
        for i in range(chb):
            os_.append(pltpu.async_copy(
                rows_v.at[pl.ds(i * S, S)], out_hbm.at[b0 + i], osem))
        for h in os_:
            h.wait()
        return carry

    lax.fori_loop(0, bpt // chb, do_chunk, 0)


def kernel(sub_logit, dip_logit, sub_pos_samples, dip_mask):
    N, K, _ = sub_logit.shape
    S = sub_pos_samples.shape[1]
    B = 256
    asd = jnp.asarray(_ASD)
    dmaskf = dip_mask.astype(jnp.float32)
    pol, sv, fidx = pl.pallas_call(
        _policy_body,
        grid=(N // B,),
        in_specs=[
            pl.BlockSpec((B, K, K), lambda i: (i, 0, 0)),
            pl.BlockSpec((B, 3), lambda i: (i, 0)),
            pl.BlockSpec((B, S), lambda i: (i, 0)),
            pl.BlockSpec((B, 3), lambda i: (i, 0)),
            pl.BlockSpec((K, K), lambda i: (0, 0)),
        ],
        out_specs=[
            pl.BlockSpec((B, K, K), lambda i: (i, 0, 0)),
            pl.BlockSpec((B, 3), lambda i: (i, 0)),
            pl.BlockSpec((B, S), lambda i: (i, 0)),
        ],
        out_shape=[
            jax.ShapeDtypeStruct((N, K, K), jnp.float32),
            jax.ShapeDtypeStruct((N, 3), jnp.float32),
            jax.ShapeDtypeStruct((N, S), jnp.int32),
        ],
    )(sub_logit, dip_logit, sub_pos_samples, dmaskf, asd)

    table16 = jnp.pad(pol.reshape(N * K, K), ((0, 0), (0, 16 - K)))

    rows = N * S
    bpt = N // _NW
    chb = 16
    mesh = plsc.VectorSubcoreMesh(core_axis_name="c", subcore_axis_name="s",
                                  num_cores=_NC, num_subcores=_NS)
    samp16 = pl.kernel(
        functools.partial(_sc_gather_body, bpt, chb, S),
        out_type=jax.ShapeDtypeStruct((N, S, 16), jnp.float32),
        mesh=mesh,
        scratch_types=[
            pltpu.VMEM((chb * S,), jnp.int32),
            pltpu.VMEM((chb * S, 16), jnp.float32),
            pltpu.SemaphoreType.DMA,
            pltpu.SemaphoreType.DMA,
        ],
        compiler_params=pltpu.CompilerParams(use_tc_tiling_on_sc=False),
    )(table16, fidx.reshape(rows))
    return (pol, jnp.maximum(samp16[:, :, :K], 0.0), sv)


# speedup vs baseline: 1.6881x; 1.4445x over previous
import functools

import jax
import jax.numpy as jnp
import numpy as np
from jax import lax
from jax.experimental import pallas as pl
from jax.experimental.pallas import tpu as pltpu
from jax.experimental.pallas import tpu_sc as plsc

_ASD = np.array([
    [0,1,1,0,0,0,0,0,0,0,0,0,0],
    [0,0,0,1,1,0,0,0,0,0,0,0,0],
    [0,0,0,0,1,1,0,0,0,0,0,0,0],
    [0,0,0,0,0,0,1,1,0,0,0,0,0],
    [0,0,0,0,0,0,0,1,1,0,0,0,0],
    [0,0,0,0,0,0,0,0,1,1,0,0,0],
    [0,0,0,0,0,0,0,0,0,0,1,0,0],
    [0,0,0,0,0,0,0,0,0,0,1,1,0],
    [0,0,0,0,0,0,0,0,0,0,0,1,1],
    [0,0,0,0,0,0,0,0,0,0,0,0,1],
    [0,0,0,0,0,0,0,0,0,0,1,0,0],
    [0,0,0,0,0,0,0,0,0,0,0,1,0],
    [0,0,0,0,0,0,0,0,0,0,0,0,1]], dtype=np.float32)

_NC = 2
_NS = 16
_NW = _NC * _NS


def _policy_body(sub_ref, dip_ref, idx_ref, dmask_ref, asd_ref,
                 pol_ref, sv_ref, fidx_ref):
    B = sub_ref.shape[0]
    l = sub_ref[...]
    row = lax.broadcasted_iota(jnp.int32, l.shape, 1)
    l = jnp.where(row == 0, 1.0, l)
    m = asd_ref[...] > 0.0
    e = jnp.where(m[None, :, :], jnp.exp(l), 0.0)
    pol_ref[...] = e / jnp.sum(e, axis=2, keepdims=True)

    d = dip_ref[...]
    de = jnp.where(dmask_ref[...] > 0.0, jnp.exp(d), 0.0)
    sv_ref[...] = de / jnp.sum(de, axis=1, keepdims=True)

    idx = idx_ref[...]
    b_global = pl.program_id(0) * B + lax.broadcasted_iota(jnp.int32, idx.shape, 0)
    fidx_ref[...] = idx + 13 * b_global


def _sc_gather_body(bpt, chb, S, table_hbm, fidx_hbm, out_hbm,
                    idx_v, rows_v, sem, osem):
    wid = lax.axis_index("s") * _NC + lax.axis_index("c")
    b00 = wid * bpt
    rows_chunk = chb * S

    def do_chunk(g, carry):
        b0 = b00 + g * chb
        base = b0 * S
        pltpu.sync_copy(fidx_hbm.at[pl.ds(base, rows_chunk)], idx_v)
        hs = []
        for j in range(rows_chunk // 128):
            hs.append(pltpu.async_copy(
                table_hbm.at[idx_v.at[pl.ds(j * 128, 128)]],
                rows_v.at[pl.ds(j * 128, 128)], sem))
        for h in hs:
            h.wait()
        os_ = []
        for i in range(chb):
            os_.append(pltpu.async_copy(
                rows_v.at[pl.ds(i * S, S)], out_hbm.at[b0 + i], osem))
        for h in os_:
            h.wait()
        return carry

    lax.fori_loop(0, bpt // chb, do_chunk, 0)


def kernel(sub_logit, dip_logit, sub_pos_samples, dip_mask):
    N, K, _ = sub_logit.shape
    S = sub_pos_samples.shape[1]
    B = 256
    asd = jnp.asarray(_ASD)
    dmaskf = dip_mask.astype(jnp.float32)
    pol, sv, fidx = pl.pallas_call(
        _policy_body,
        grid=(N // B,),
        in_specs=[
            pl.BlockSpec((B, K, K), lambda i: (i, 0, 0)),
            pl.BlockSpec((B, 3), lambda i: (i, 0)),
            pl.BlockSpec((B, S), lambda i: (i, 0)),
            pl.BlockSpec((B, 3), lambda i: (i, 0)),
            pl.BlockSpec((K, K), lambda i: (0, 0)),
        ],
        out_specs=[
            pl.BlockSpec((B, K, K), lambda i: (i, 0, 0)),
            pl.BlockSpec((B, 3), lambda i: (i, 0)),
            pl.BlockSpec((B, S), lambda i: (i, 0)),
        ],
        out_shape=[
            jax.ShapeDtypeStruct((N, K, K), jnp.float32),
            jax.ShapeDtypeStruct((N, 3), jnp.float32),
            jax.ShapeDtypeStruct((N, S), jnp.int32),
        ],
    )(sub_logit, dip_logit, sub_pos_samples, dmaskf, asd)

    table16 = jnp.pad(pol, ((0, 0), (0, 0), (0, 16 - K))).reshape(N * K, 16)

    rows = N * S
    bpt = N // _NW
    chb = 16
    mesh = plsc.VectorSubcoreMesh(core_axis_name="c", subcore_axis_name="s",
                                  num_cores=_NC, num_subcores=_NS)
    samp16 = pl.kernel(
        functools.partial(_sc_gather_body, bpt, chb, S),
        out_type=jax.ShapeDtypeStruct((N, S, 16), jnp.float32),
        mesh=mesh,
        scratch_types=[
            pltpu.VMEM((chb * S,), jnp.int32),
            pltpu.VMEM((chb * S, 16), jnp.float32),
            pltpu.SemaphoreType.DMA,
            pltpu.SemaphoreType.DMA,
        ],
        compiler_params=pltpu.CompilerParams(use_tc_tiling_on_sc=False),
    )(table16, fidx.reshape(rows))
    return (pol, samp16[:, :, :K], sv)


# speedup vs baseline: 1.6888x; 1.0004x over previous
import functools

import jax
import jax.numpy as jnp
import numpy as np
from jax import lax
from jax.experimental import pallas as pl
from jax.experimental.pallas import tpu as pltpu
from jax.experimental.pallas import tpu_sc as plsc

_ASD = np.array([
    [0,1,1,0,0,0,0,0,0,0,0,0,0],
    [0,0,0,1,1,0,0,0,0,0,0,0,0],
    [0,0,0,0,1,1,0,0,0,0,0,0,0],
    [0,0,0,0,0,0,1,1,0,0,0,0,0],
    [0,0,0,0,0,0,0,1,1,0,0,0,0],
    [0,0,0,0,0,0,0,0,1,1,0,0,0],
    [0,0,0,0,0,0,0,0,0,0,1,0,0],
    [0,0,0,0,0,0,0,0,0,0,1,1,0],
    [0,0,0,0,0,0,0,0,0,0,0,1,1],
    [0,0,0,0,0,0,0,0,0,0,0,0,1],
    [0,0,0,0,0,0,0,0,0,0,1,0,0],
    [0,0,0,0,0,0,0,0,0,0,0,1,0],
    [0,0,0,0,0,0,0,0,0,0,0,0,1]], dtype=np.float32)

_NC = 2
_NS = 16
_NW = _NC * _NS


def _policy_body(sub_ref, dip_ref, idx_ref, dmask_ref, asd_ref,
                 pol_ref, sv_ref, fidx_ref):
    B = sub_ref.shape[0]
    l = sub_ref[...]
    row = lax.broadcasted_iota(jnp.int32, l.shape, 1)
    l = jnp.where(row == 0, 1.0, l)
    m = asd_ref[...] > 0.0
    e = jnp.where(m[None, :, :], jnp.exp(l), 0.0)
    pol_ref[...] = e / jnp.sum(e, axis=2, keepdims=True)

    d = dip_ref[...]
    de = jnp.where(dmask_ref[...] > 0.0, jnp.exp(d), 0.0)
    sv_ref[...] = de / jnp.sum(de, axis=1, keepdims=True)

    idx = idx_ref[...]
    b_global = pl.program_id(0) * B + lax.broadcasted_iota(jnp.int32, idx.shape, 0)
    fidx_ref[...] = idx + 13 * b_global


def _sc_gather_body(bpt, chb, S, table_hbm, fidx_hbm, out_hbm,
                    idx_v, rows_v, sem, osem):
    wid = lax.axis_index("s") * _NC + lax.axis_index("c")
    b00 = wid * bpt
    rows_chunk = chb * S

    def do_chunk(g, carry):
        b0 = b00 + g * chb
        base = b0 * S
        pltpu.sync_copy(fidx_hbm.at[pl.ds(base // 128, rows_chunk // 128)],
                        idx_v)
        hs = []
        for j in range(rows_chunk // 128):
            hs.append(pltpu.async_copy(
                table_hbm.at[idx_v.at[j]],
                rows_v.at[pl.ds(j * 128, 128)], sem))
        for h in hs:
            h.wait()
        os_ = []
        for i in range(chb):
            os_.append(pltpu.async_copy(
                rows_v.at[pl.ds(i * S, S)], out_hbm.at[b0 + i], osem))
        for h in os_:
            h.wait()
        return carry

    lax.fori_loop(0, bpt // chb, do_chunk, 0)


def kernel(sub_logit, dip_logit, sub_pos_samples, dip_mask):
    N, K, _ = sub_logit.shape
    S = sub_pos_samples.shape[1]
    B = 256
    asd = jnp.asarray(_ASD)
    dmaskf = dip_mask.astype(jnp.float32)
    pol, sv, fidx = pl.pallas_call(
        _policy_body,
        grid=(N // B,),
        in_specs=[
            pl.BlockSpec((B, K, K), lambda i: (i, 0, 0)),
            pl.BlockSpec((B, 3), lambda i: (i, 0)),
            pl.BlockSpec((B, S), lambda i: (i, 0)),
            pl.BlockSpec((B, 3), lambda i: (i, 0)),
            pl.BlockSpec((K, K), lambda i: (0, 0)),
        ],
        out_specs=[
            pl.BlockSpec((B, K, K), lambda i: (i, 0, 0)),
            pl.BlockSpec((B, 3), lambda i: (i, 0)),
            pl.BlockSpec((B, S), lambda i: (i, 0)),
        ],
        out_shape=[
            jax.ShapeDtypeStruct((N, K, K), jnp.float32),
            jax.ShapeDtypeStruct((N, 3), jnp.float32),
            jax.ShapeDtypeStruct((N, S), jnp.int32),
        ],
    )(sub_logit, dip_logit, sub_pos_samples, dmaskf, asd)

    table16 = jnp.pad(pol, ((0, 0), (0, 0), (0, 16 - K))).reshape(N * K, 16)

    rows = N * S
    bpt = N // _NW
    chb = 16
    mesh = plsc.VectorSubcoreMesh(core_axis_name="c", subcore_axis_name="s",
                                  num_cores=_NC, num_subcores=_NS)
    samp16 = pl.kernel(
        functools.partial(_sc_gather_body, bpt, chb, S),
        out_type=jax.ShapeDtypeStruct((N, S, 16), jnp.float32),
        mesh=mesh,
        scratch_types=[
            pltpu.VMEM((chb * S // 128, 128), jnp.int32),
            pltpu.VMEM((chb * S, 16), jnp.float32),
            pltpu.SemaphoreType.DMA,
            pltpu.SemaphoreType.DMA,
        ],
        compiler_params=pltpu.CompilerParams(use_tc_tiling_on_sc=False),
    )(table16, fidx.reshape(rows // 128, 128))
    return (pol, samp16[:, :, :K], sv)
